# baseline (speedup 1.0000x reference)
import functools

import jax
import jax.numpy as jnp
from jax import lax
from jax.experimental import pallas as pl
from jax.experimental.pallas import tpu as pltpu
from jax.experimental.pallas import tpu_sc as plsc

NC = 2
NS = 16
NW = NC * NS
B = 16384
D = 32
V = 1000000
PACK = 4
BPW = B // NW
TC_CHUNK = 8192
BAND = TC_CHUNK // PACK
NSTEP = pl.cdiv(V, TC_CHUNK)
VP = NSTEP * BAND


def _transpose_kernel(x_ref, eye_ref, o_ref):
    x = x_ref[...]
    eye = eye_ref[...]
    bands = [
        jax.lax.dot_general(
            x[:, a * BAND:(a + 1) * BAND], eye,
            dimension_numbers=(((0,), (0,)), ((), ())),
            preferred_element_type=jnp.float32)
        for a in range(PACK)
    ]
    o_ref[...] = jnp.concatenate(bands, axis=1)


def _pack_table(tabT, eye):
    return pl.pallas_call(
        _transpose_kernel,
        grid=(NSTEP,),
        in_specs=[
            pl.BlockSpec((D, TC_CHUNK), lambda i: (0, i)),
            pl.BlockSpec((D, D), lambda i: (0, 0)),
        ],
        out_specs=pl.BlockSpec((BAND, D * PACK), lambda i: (i, 0)),
        out_shape=jax.ShapeDtypeStruct((VP, D * PACK), jnp.float32),
        compiler_params=pltpu.CompilerParams(
            dimension_semantics=("arbitrary",)),
    )(tabT, eye)


def _gather_sc(user, product, t4u, t4p):
    mesh = plsc.VectorSubcoreMesh(core_axis_name="c", subcore_axis_name="s")

    @functools.partial(
        pl.kernel,
        mesh=mesh,
        out_type=[
            jax.ShapeDtypeStruct((B, D * PACK), jnp.float32),
            jax.ShapeDtypeStruct((B, D * PACK), jnp.float32),
        ],
        scratch_types=[
            pltpu.VMEM((BPW,), jnp.int32),
            pltpu.VMEM((BPW,), jnp.int32),
            pltpu.VMEM((BPW, D * PACK), jnp.float32),
            pltpu.SemaphoreType.DMA,
        ],
    )
    def k(uidx_hbm, pidx_hbm, ut_hbm, pt_hbm, gu_hbm, gp_hbm,
          idx_v, j_v, rows_v, sem):
        wid = lax.axis_index("s") * NC + lax.axis_index("c")
        base = wid * BPW

        pltpu.sync_copy(uidx_hbm.at[pl.ds(base, BPW)], idx_v)

        @pl.loop(0, BPW, step=16)
        def _(i):
            u = idx_v.at[pl.ds(i, 16)][...]
            j_v.at[pl.ds(i, 16)][...] = ((u >> 13) << 11) | (u & 2047)

        pltpu.async_copy(ut_hbm.at[j_v], rows_v, sem).wait()
        pltpu.sync_copy(rows_v, gu_hbm.at[pl.ds(base, BPW)])

        pltpu.sync_copy(pidx_hbm.at[pl.ds(base, BPW)], idx_v)

        @pl.loop(0, BPW, step=16)
        def _(i):
            u = idx_v.at[pl.ds(i, 16)][...]
            j_v.at[pl.ds(i, 16)][...] = ((u >> 13) << 11) | (u & 2047)

        pltpu.async_copy(pt_hbm.at[j_v], rows_v, sem).wait()
        pltpu.sync_copy(rows_v, gp_hbm.at[pl.ds(base, BPW)])

    return k(user, product, t4u, t4p)


def _extract(g, amod):
    out = jnp.zeros((g.shape[0], D), jnp.float32)
    for a in range(PACK):
        m = (amod == a).astype(jnp.float32)
        out = out + m * g[:, a * D:(a + 1) * D]
    return out


def _mlp_kernel(gu_ref, gp_ref, ui_ref, pi_ref, w1u_ref, w1p_ref, b1_ref,
                w2_ref, b2_ref, w3_ref, b3_ref, out_ref):
    ue = _extract(gu_ref[...], (ui_ref[...] >> 11) & 3)
    pe = _extract(gp_ref[...], (pi_ref[...] >> 11) & 3)
    h = jnp.dot(ue, w1u_ref[...], preferred_element_type=jnp.float32)
    h = h + jnp.dot(pe, w1p_ref[...], preferred_element_type=jnp.float32)
    h = jnp.maximum(h + b1_ref[...], 0.0)
    h = jnp.maximum(
        jnp.dot(h, w2_ref[...], preferred_element_type=jnp.float32)
        + b2_ref[...], 0.0)
    o = jnp.dot(h, w3_ref[...], preferred_element_type=jnp.float32) + b3_ref[...]
    out_ref[...] = jax.nn.sigmoid(o)


def _mlp_tc(gu, gp, user, product, W1, b1, W2, b2, W3, b3):
    W1uT = W1[:, :D].T
    W1pT = W1[:, D:].T
    W2T = W2.T
    W3T = W3.T
    b1r = b1.reshape(1, 128)
    b2r = b2.reshape(1, 64)
    b3r = b3.reshape(1, 1)
    ui = user.reshape(B, 1)
    pi = product.reshape(B, 1)
    BB = 2048
    grid = (B // BB,)
    return pl.pallas_call(
        _mlp_kernel,
        grid=grid,
        in_specs=[
            pl.BlockSpec((BB, D * PACK), lambda i: (i, 0)),
            pl.BlockSpec((BB, D * PACK), lambda i: (i, 0)),
            pl.BlockSpec((BB, 1), lambda i: (i, 0)),
            pl.BlockSpec((BB, 1), lambda i: (i, 0)),
            pl.BlockSpec((D, 128), lambda i: (0, 0)),
            pl.BlockSpec((D, 128), lambda i: (0, 0)),
            pl.BlockSpec((1, 128), lambda i: (0, 0)),
            pl.BlockSpec((128, 64), lambda i: (0, 0)),
            pl.BlockSpec((1, 64), lambda i: (0, 0)),
            pl.BlockSpec((64, 1), lambda i: (0, 0)),
            pl.BlockSpec((1, 1), lambda i: (0, 0)),
        ],
        out_specs=pl.BlockSpec((BB, 1), lambda i: (i, 0)),
        out_shape=jax.ShapeDtypeStruct((B, 1), jnp.float32),
        compiler_params=pltpu.CompilerParams(
            dimension_semantics=("arbitrary",)),
    )(gu, gp, ui, pi, W1uT, W1pT, b1r, W2T, b2r, W3T, b3r)


def kernel(user, product, user_emb, prod_emb, W1, b1, W2, b2, W3, b3):
    eye = jnp.eye(D, dtype=jnp.float32)
    t4u = _pack_table(user_emb.T, eye)
    t4p = _pack_table(prod_emb.T, eye)
    gu, gp = _gather_sc(user, product, t4u, t4p)
    return _mlp_tc(gu, gp, user, product, W1, b1, W2, b2, W3, b3)


# speedup vs baseline: 1.2688x; 1.2688x over previous
import functools

import jax
import jax.numpy as jnp
from jax import lax
from jax.experimental import pallas as pl
from jax.experimental.pallas import tpu as pltpu
from jax.experimental.pallas import tpu_sc as plsc

NC = 2
NS = 16
NW = NC * NS
B = 16384
D = 32
V = 1000000
PACK = 4
BPW = B // NW
TC_CHUNK = 8192
BAND = TC_CHUNK // PACK
NSTEP = pl.cdiv(V, TC_CHUNK)
VP = NSTEP * BAND


def _transpose_kernel(x_ref, eyes_ref, o_ref):
    x = x_ref[...]
    acc = None
    for a in range(PACK):
        y = jax.lax.dot_general(
            x[:, a * BAND:(a + 1) * BAND],
            eyes_ref[:, a * (D * PACK):(a + 1) * (D * PACK)],
            dimension_numbers=(((0,), (0,)), ((), ())),
            preferred_element_type=jnp.float32)
        acc = y if acc is None else acc + y
    o_ref[...] = acc


def _pack_table(tabT, eyes):
    return pl.pallas_call(
        _transpose_kernel,
        grid=(NSTEP,),
        in_specs=[
            pl.BlockSpec((D, TC_CHUNK), lambda i: (0, i)),
            pl.BlockSpec((D, D * PACK * PACK), lambda i: (0, 0)),
        ],
        out_specs=pl.BlockSpec((BAND, D * PACK), lambda i: (i, 0)),
        out_shape=jax.ShapeDtypeStruct((VP, D * PACK), jnp.float32),
        compiler_params=pltpu.CompilerParams(
            dimension_semantics=("arbitrary",)),
    )(tabT, eyes)


def _gather_sc(user, product, t4u, t4p):
    mesh = plsc.VectorSubcoreMesh(core_axis_name="c", subcore_axis_name="s")

    @functools.partial(
        pl.kernel,
        mesh=mesh,
        out_type=[
            jax.ShapeDtypeStruct((B, D * PACK), jnp.float32),
            jax.ShapeDtypeStruct((B, D * PACK), jnp.float32),
        ],
        scratch_types=[
            pltpu.VMEM((BPW,), jnp.int32),
            pltpu.VMEM((BPW,), jnp.int32),
            pltpu.VMEM((BPW, D * PACK), jnp.float32),
            pltpu.SemaphoreType.DMA,
        ],
    )
    def k(uidx_hbm, pidx_hbm, ut_hbm, pt_hbm, gu_hbm, gp_hbm,
          idx_v, j_v, rows_v, sem):
        wid = lax.axis_index("s") * NC + lax.axis_index("c")
        base = wid * BPW

        pltpu.sync_copy(uidx_hbm.at[pl.ds(base, BPW)], idx_v)

        @pl.loop(0, BPW, step=16)
        def _(i):
            u = idx_v.at[pl.ds(i, 16)][...]
            j_v.at[pl.ds(i, 16)][...] = ((u >> 13) << 11) | (u & 2047)

        pltpu.async_copy(ut_hbm.at[j_v], rows_v, sem).wait()
        pltpu.sync_copy(rows_v, gu_hbm.at[pl.ds(base, BPW)])

        pltpu.sync_copy(pidx_hbm.at[pl.ds(base, BPW)], idx_v)

        @pl.loop(0, BPW, step=16)
        def _(i):
            u = idx_v.at[pl.ds(i, 16)][...]
            j_v.at[pl.ds(i, 16)][...] = ((u >> 13) << 11) | (u & 2047)

        pltpu.async_copy(pt_hbm.at[j_v], rows_v, sem).wait()
        pltpu.sync_copy(rows_v, gp_hbm.at[pl.ds(base, BPW)])

    return k(user, product, t4u, t4p)


def _extract(g, amod):
    out = jnp.zeros((g.shape[0], D), jnp.float32)
    for a in range(PACK):
        m = (amod == a).astype(jnp.float32)
        out = out + m * g[:, a * D:(a + 1) * D]
    return out


def _mlp_kernel(gu_ref, gp_ref, ui_ref, pi_ref, w1u_ref, w1p_ref, b1_ref,
                w2_ref, b2_ref, w3_ref, b3_ref, out_ref):
    ue = _extract(gu_ref[...], (ui_ref[...] >> 11) & 3)
    pe = _extract(gp_ref[...], (pi_ref[...] >> 11) & 3)
    h = jnp.dot(ue, w1u_ref[...], preferred_element_type=jnp.float32)
    h = h + jnp.dot(pe, w1p_ref[...], preferred_element_type=jnp.float32)
    h = jnp.maximum(h + b1_ref[...], 0.0)
    h = jnp.maximum(
        jnp.dot(h, w2_ref[...], preferred_element_type=jnp.float32)
        + b2_ref[...], 0.0)
    o = jnp.dot(h, w3_ref[...], preferred_element_type=jnp.float32) + b3_ref[...]
    out_ref[...] = jax.nn.sigmoid(o)


def _mlp_tc(gu, gp, user, product, W1, b1, W2, b2, W3, b3):
    W1uT = W1[:, :D].T
    W1pT = W1[:, D:].T
    W2T = W2.T
    W3T = W3.T
    b1r = b1.reshape(1, 128)
    b2r = b2.reshape(1, 64)
    b3r = b3.reshape(1, 1)
    ui = user.reshape(B, 1)
    pi = product.reshape(B, 1)
    BB = 2048
    grid = (B // BB,)
    return pl.pallas_call(
        _mlp_kernel,
        grid=grid,
        in_specs=[
            pl.BlockSpec((BB, D * PACK), lambda i: (i, 0)),
            pl.BlockSpec((BB, D * PACK), lambda i: (i, 0)),
            pl.BlockSpec((BB, 1), lambda i: (i, 0)),
            pl.BlockSpec((BB, 1), lambda i: (i, 0)),
            pl.BlockSpec((D, 128), lambda i: (0, 0)),
            pl.BlockSpec((D, 128), lambda i: (0, 0)),
            pl.BlockSpec((1, 128), lambda i: (0, 0)),
            pl.BlockSpec((128, 64), lambda i: (0, 0)),
            pl.BlockSpec((1, 64), lambda i: (0, 0)),
            pl.BlockSpec((64, 1), lambda i: (0, 0)),
            pl.BlockSpec((1, 1), lambda i: (0, 0)),
        ],
        out_specs=pl.BlockSpec((BB, 1), lambda i: (i, 0)),
        out_shape=jax.ShapeDtypeStruct((B, 1), jnp.float32),
        compiler_params=pltpu.CompilerParams(
            dimension_semantics=("arbitrary",)),
    )(gu, gp, ui, pi, W1uT, W1pT, b1r, W2T, b2r, W3T, b3r)


def kernel(user, product, user_emb, prod_emb, W1, b1, W2, b2, W3, b3):
    eye = jnp.eye(D, dtype=jnp.float32)
    eyes = jnp.concatenate(
        [jnp.pad(eye, ((0, 0), (a * D, (PACK - 1 - a) * D)))
         for a in range(PACK)], axis=1)
    t4u = _pack_table(user_emb.T, eyes)
    t4p = _pack_table(prod_emb.T, eyes)
    gu, gp = _gather_sc(user, product, t4u, t4p)
    return _mlp_tc(gu, gp, user, product, W1, b1, W2, b2, W3, b3)


# speedup vs baseline: 1.4632x; 1.1532x over previous
import functools

import jax
import jax.numpy as jnp
from jax import lax
from jax.experimental import pallas as pl
from jax.experimental.pallas import tpu as pltpu
from jax.experimental.pallas import tpu_sc as plsc

NC = 2
NS = 16
NW = NC * NS
B = 16384
D = 32
V = 1000000
PACK = 4
BPW = B // NW
TC_CHUNK = 8192
BAND = TC_CHUNK // PACK
NSTEP = pl.cdiv(V, TC_CHUNK)
VP = NSTEP * BAND


def _transpose_kernel(x_ref, eyes_ref, o_ref):
    x = x_ref[...].astype(jnp.bfloat16)
    acc = None
    for a in range(PACK):
        y = jax.lax.dot_general(
            x[:, a * BAND:(a + 1) * BAND],
            eyes_ref[:, a * (D * PACK):(a + 1) * (D * PACK)],
            dimension_numbers=(((0,), (0,)), ((), ())),
            preferred_element_type=jnp.float32)
        acc = y if acc is None else acc + y
    o_ref[...] = pltpu.bitcast(acc.astype(jnp.bfloat16), jnp.float32)


def _pack_table(tabT, eyes):
    return pl.pallas_call(
        _transpose_kernel,
        grid=(NSTEP,),
        in_specs=[
            pl.BlockSpec((D, TC_CHUNK), lambda i: (0, i)),
            pl.BlockSpec((D, D * PACK * PACK), lambda i: (0, 0)),
        ],
        out_specs=pl.BlockSpec((BAND // 2, D * PACK), lambda i: (i, 0)),
        out_shape=jax.ShapeDtypeStruct((VP // 2, D * PACK), jnp.float32),
        compiler_params=pltpu.CompilerParams(
            dimension_semantics=("arbitrary",)),
    )(tabT, eyes)


def _gather_sc(user, product, t4u, t4p):
    mesh = plsc.VectorSubcoreMesh(core_axis_name="c", subcore_axis_name="s")

    @functools.partial(
        pl.kernel,
        mesh=mesh,
        out_type=[
            jax.ShapeDtypeStruct((B, D * PACK), jnp.float32),
            jax.ShapeDtypeStruct((B, D * PACK), jnp.float32),
        ],
        scratch_types=[
            pltpu.VMEM((BPW,), jnp.int32),
            pltpu.VMEM((BPW,), jnp.int32),
            pltpu.VMEM((BPW, D * PACK), jnp.float32),
            pltpu.SemaphoreType.DMA,
        ],
    )
    def k(uidx_hbm, pidx_hbm, ut_hbm, pt_hbm, gu_hbm, gp_hbm,
          idx_v, j_v, rows_v, sem):
        wid = lax.axis_index("s") * NC + lax.axis_index("c")
        base = wid * BPW

        pltpu.sync_copy(uidx_hbm.at[pl.ds(base, BPW)], idx_v)

        @pl.loop(0, BPW, step=16)
        def _(i):
            u = idx_v.at[pl.ds(i, 16)][...]
            j_v.at[pl.ds(i, 16)][...] = (((u >> 13) << 11) | (u & 2047)) >> 1

        pltpu.async_copy(ut_hbm.at[j_v], rows_v, sem).wait()
        pltpu.sync_copy(rows_v, gu_hbm.at[pl.ds(base, BPW)])

        pltpu.sync_copy(pidx_hbm.at[pl.ds(base, BPW)], idx_v)

        @pl.loop(0, BPW, step=16)
        def _(i):
            u = idx_v.at[pl.ds(i, 16)][...]
            j_v.at[pl.ds(i, 16)][...] = (((u >> 13) << 11) | (u & 2047)) >> 1

        pltpu.async_copy(pt_hbm.at[j_v], rows_v, sem).wait()
        pltpu.sync_copy(rows_v, gp_hbm.at[pl.ds(base, BPW)])

    return k(user, product, t4u, t4p)


def _extract(g, idx):
    gi = jax.lax.bitcast_convert_type(g, jnp.int32)
    g0 = jax.lax.bitcast_convert_type(gi << 16, jnp.float32)
    g1 = jax.lax.bitcast_convert_type(
        gi & jnp.int32(-65536), jnp.float32)
    amod = (idx >> 11) & 3
    pmod = idx & 1
    out = jnp.zeros((g.shape[0], D), jnp.float32)
    for p, gf in ((0, g0), (1, g1)):
        for a in range(PACK):
            m = ((amod == a) & (pmod == p)).astype(jnp.float32)
            out = out + m * gf[:, a * D:(a + 1) * D]
    return out


def _mlp_kernel(gu_ref, gp_ref, ui_ref, pi_ref,
                w1u_ref, w1p_ref, b1_ref,
                w2_ref, b2_ref, w3_ref, b3_ref, out_ref):
    ue = _extract(gu_ref[...], ui_ref[...])
    pe = _extract(gp_ref[...], pi_ref[...])
    h = jnp.dot(ue, w1u_ref[...], preferred_element_type=jnp.float32)
    h = h + jnp.dot(pe, w1p_ref[...], preferred_element_type=jnp.float32)
    h = jnp.maximum(h + b1_ref[...], 0.0)
    h = jnp.maximum(
        jnp.dot(h, w2_ref[...], preferred_element_type=jnp.float32)
        + b2_ref[...], 0.0)
    o = jnp.dot(h, w3_ref[...], preferred_element_type=jnp.float32) + b3_ref[...]
    out_ref[...] = jax.nn.sigmoid(o)


def _mlp_tc(gu, gp, user, product, W1, b1, W2, b2, W3, b3):
    W1uT = W1[:, :D].T
    W1pT = W1[:, D:].T
    W2T = W2.T
    W3T = W3.T
    b1r = b1.reshape(1, 128)
    b2r = b2.reshape(1, 64)
    b3r = b3.reshape(1, 1)
    ui = user.reshape(B, 1)
    pi = product.reshape(B, 1)
    BB = 2048
    grid = (B // BB,)
    return pl.pallas_call(
        _mlp_kernel,
        grid=grid,
        in_specs=[
            pl.BlockSpec((BB, D * PACK), lambda i: (i, 0)),
            pl.BlockSpec((BB, D * PACK), lambda i: (i, 0)),
            pl.BlockSpec((BB, 1), lambda i: (i, 0)),
            pl.BlockSpec((BB, 1), lambda i: (i, 0)),
            pl.BlockSpec((D, 128), lambda i: (0, 0)),
            pl.BlockSpec((D, 128), lambda i: (0, 0)),
            pl.BlockSpec((1, 128), lambda i: (0, 0)),
            pl.BlockSpec((128, 64), lambda i: (0, 0)),
            pl.BlockSpec((1, 64), lambda i: (0, 0)),
            pl.BlockSpec((64, 1), lambda i: (0, 0)),
            pl.BlockSpec((1, 1), lambda i: (0, 0)),
        ],
        out_specs=pl.BlockSpec((BB, 1), lambda i: (i, 0)),
        out_shape=jax.ShapeDtypeStruct((B, 1), jnp.float32),
        compiler_params=pltpu.CompilerParams(
            dimension_semantics=("arbitrary",)),
    )(gu, gp, ui, pi, W1uT, W1pT, b1r, W2T, b2r, W3T, b3r)


def kernel(user, product, user_emb, prod_emb, W1, b1, W2, b2, W3, b3):
    eye = jnp.eye(D, dtype=jnp.float32)
    eyes = jnp.concatenate(
        [jnp.pad(eye, ((0, 0), (a * D, (PACK - 1 - a) * D)))
         for a in range(PACK)], axis=1).astype(jnp.bfloat16)
    t4u = _pack_table(user_emb.T, eyes)
    t4p = _pack_table(prod_emb.T, eyes)
    gu, gp = _gather_sc(user, product, t4u, t4p)
    return _mlp_tc(gu, gp, user, product, W1, b1, W2, b2, W3, b3)


# speedup vs baseline: 1.9786x; 1.3523x over previous
import functools

import jax
import jax.numpy as jnp
from jax import lax
from jax.experimental import pallas as pl
from jax.experimental.pallas import tpu as pltpu
from jax.experimental.pallas import tpu_sc as plsc

NC = 2
NS = 16
NW = NC * NS
B = 16384
D = 32
V = 1000000
PACK = 4
BPW = B // NW
TC_CHUNK = 16384
BAND = TC_CHUNK // PACK
NSTEP = pl.cdiv(V, TC_CHUNK)
VP = NSTEP * BAND
BANDBITS = BAND.bit_length() - 1
STEPBITS = TC_CHUNK.bit_length() - 1


def _transpose_kernel(x_ref, eyes_ref, o_ref):
    x = x_ref[...].astype(jnp.bfloat16)
    acc = None
    for a in range(PACK):
        y = jax.lax.dot_general(
            x[:, a * BAND:(a + 1) * BAND],
            eyes_ref[:, a * (D * PACK):(a + 1) * (D * PACK)],
            dimension_numbers=(((0,), (0,)), ((), ())),
            preferred_element_type=jnp.float32)
        acc = y if acc is None else acc + y
    o_ref[...] = pltpu.bitcast(acc.astype(jnp.bfloat16), jnp.float32)


def _pack_table(tabT, eyes):
    return pl.pallas_call(
        _transpose_kernel,
        grid=(NSTEP,),
        in_specs=[
            pl.BlockSpec((D, TC_CHUNK), lambda i: (0, i)),
            pl.BlockSpec((D, D * PACK * PACK), lambda i: (0, 0)),
        ],
        out_specs=pl.BlockSpec((BAND // 2, D * PACK), lambda i: (i, 0)),
        out_shape=jax.ShapeDtypeStruct((VP // 2, D * PACK), jnp.float32),
        compiler_params=pltpu.CompilerParams(
            dimension_semantics=("arbitrary",)),
    )(tabT, eyes)


def _gather_sc(idx, t4):
    mesh = plsc.VectorSubcoreMesh(core_axis_name="c", subcore_axis_name="s")

    @functools.partial(
        pl.kernel,
        mesh=mesh,
        out_type=jax.ShapeDtypeStruct((B, D * PACK), jnp.float32),
        scratch_types=[
            pltpu.VMEM((BPW,), jnp.int32),
            pltpu.VMEM((BPW,), jnp.int32),
            pltpu.VMEM((BPW, D * PACK), jnp.float32),
            pltpu.SemaphoreType.DMA,
        ],
    )
    def k(idx_hbm, t_hbm, g_hbm, idx_v, j_v, rows_v, sem):
        wid = lax.axis_index("s") * NC + lax.axis_index("c")
        base = wid * BPW

        pltpu.sync_copy(idx_hbm.at[pl.ds(base, BPW)], idx_v)

        @pl.loop(0, BPW, step=16)
        def _(i):
            u = idx_v.at[pl.ds(i, 16)][...]
            j_v.at[pl.ds(i, 16)][...] = (
                ((u >> STEPBITS) << BANDBITS) | (u & (BAND - 1))) >> 1

        pltpu.async_copy(t_hbm.at[j_v], rows_v, sem).wait()
        pltpu.sync_copy(rows_v, g_hbm.at[pl.ds(base, BPW)])

    return k(idx, t4)


def _extract(g, idx):
    gi = jax.lax.bitcast_convert_type(g, jnp.int32)
    podd = (idx & 1) == 1
    gsel = jnp.where(podd, gi & jnp.int32(-65536), gi << 16)
    gf = jax.lax.bitcast_convert_type(gsel, jnp.float32).astype(jnp.bfloat16)
    amod = (idx >> BANDBITS) & 3
    out = jnp.zeros((g.shape[0], D), jnp.bfloat16)
    for a in range(PACK):
        m = (amod == a).astype(jnp.bfloat16)
        out = out + m * gf[:, a * D:(a + 1) * D]
    return out


def _mlp_kernel(gu_ref, gp_ref, ui_ref, pi_ref,
                w1u_ref, w1p_ref, b1_ref,
                w2_ref, b2_ref, w3_ref, b3_ref, out_ref):
    ue = _extract(gu_ref[...], ui_ref[...]).astype(jnp.float32)
    pe = _extract(gp_ref[...], pi_ref[...]).astype(jnp.float32)
    h = jnp.dot(ue, w1u_ref[...], preferred_element_type=jnp.float32)
    h = h + jnp.dot(pe, w1p_ref[...], preferred_element_type=jnp.float32)
    h = jnp.maximum(h + b1_ref[...], 0.0)
    h = jnp.maximum(
        jnp.dot(h, w2_ref[...], preferred_element_type=jnp.float32)
        + b2_ref[...], 0.0)
    o = jnp.dot(h, w3_ref[...], preferred_element_type=jnp.float32) + b3_ref[...]
    out_ref[...] = jax.nn.sigmoid(o)


def _mlp_tc(gu, gp, user, product, W1, b1, W2, b2, W3, b3):
    W1uT = W1[:, :D].T
    W1pT = W1[:, D:].T
    W2T = W2.T
    W3T = W3.T
    b1r = b1.reshape(1, 128)
    b2r = b2.reshape(1, 64)
    b3r = b3.reshape(1, 1)
    ui = user.reshape(B, 1)
    pi = product.reshape(B, 1)
    BB = 2048
    grid = (B // BB,)
    return pl.pallas_call(
        _mlp_kernel,
        grid=grid,
        in_specs=[
            pl.BlockSpec((BB, D * PACK), lambda i: (i, 0)),
            pl.BlockSpec((BB, D * PACK), lambda i: (i, 0)),
            pl.BlockSpec((BB, 1), lambda i: (i, 0)),
            pl.BlockSpec((BB, 1), lambda i: (i, 0)),
            pl.BlockSpec((D, 128), lambda i: (0, 0)),
            pl.BlockSpec((D, 128), lambda i: (0, 0)),
            pl.BlockSpec((1, 128), lambda i: (0, 0)),
            pl.BlockSpec((128, 64), lambda i: (0, 0)),
            pl.BlockSpec((1, 64), lambda i: (0, 0)),
            pl.BlockSpec((64, 1), lambda i: (0, 0)),
            pl.BlockSpec((1, 1), lambda i: (0, 0)),
        ],
        out_specs=pl.BlockSpec((BB, 1), lambda i: (i, 0)),
        out_shape=jax.ShapeDtypeStruct((B, 1), jnp.float32),
        compiler_params=pltpu.CompilerParams(
            dimension_semantics=("arbitrary",)),
    )(gu, gp, ui, pi, W1uT, W1pT, b1r, W2T, b2r, W3T, b3r)


def kernel(user, product, user_emb, prod_emb, W1, b1, W2, b2, W3, b3):
    eye = jnp.eye(D, dtype=jnp.float32)
    eyes = jnp.concatenate(
        [jnp.pad(eye, ((0, 0), (a * D, (PACK - 1 - a) * D)))
         for a in range(PACK)], axis=1).astype(jnp.bfloat16)
    t4u = _pack_table(user_emb.T, eyes)
    gu = _gather_sc(user, t4u)
    t4p = _pack_table(prod_emb.T, eyes)
    gp = _gather_sc(product, t4p)
    return _mlp_tc(gu, gp, user, product, W1, b1, W2, b2, W3, b3)


# speedup vs baseline: 2.3624x; 1.1940x over previous
import functools

import jax
import jax.numpy as jnp
from jax import lax
from jax.experimental import pallas as pl
from jax.experimental.pallas import tpu as pltpu
from jax.experimental.pallas import tpu_sc as plsc

NC = 2
NS = 16
NW = NC * NS
B = 16384
D = 32
V = 1000000
PACK = 4
BPW = B // NW
TC_CHUNK = 16384
BAND = TC_CHUNK // PACK
NSTEP = pl.cdiv(V, TC_CHUNK)
VP = NSTEP * BAND
BANDBITS = BAND.bit_length() - 1
STEPBITS = TC_CHUNK.bit_length() - 1


def _transpose_kernel(x_ref, eyes_ref, o_ref):
    x = x_ref[...].astype(jnp.bfloat16)
    xs = jnp.concatenate(
        [x[:, a * BAND:(a + 1) * BAND] for a in range(PACK)], axis=0)
    acc = jax.lax.dot_general(
        xs, eyes_ref[...],
        dimension_numbers=(((0,), (0,)), ((), ())),
        preferred_element_type=jnp.float32)
    o_ref[...] = pltpu.bitcast(acc.astype(jnp.bfloat16), jnp.float32)


def _pack_table(tabT, eyes):
    return pl.pallas_call(
        _transpose_kernel,
        grid=(NSTEP,),
        in_specs=[
            pl.BlockSpec((D, TC_CHUNK), lambda i: (0, i)),
            pl.BlockSpec((D * PACK, D * PACK), lambda i: (0, 0)),
        ],
        out_specs=pl.BlockSpec((BAND // 2, D * PACK), lambda i: (i, 0)),
        out_shape=jax.ShapeDtypeStruct((VP // 2, D * PACK), jnp.float32),
        compiler_params=pltpu.CompilerParams(
            dimension_semantics=("arbitrary",)),
    )(tabT, eyes)


def _gather_sc(idx, t4):
    mesh = plsc.VectorSubcoreMesh(core_axis_name="c", subcore_axis_name="s")

    @functools.partial(
        pl.kernel,
        mesh=mesh,
        out_type=jax.ShapeDtypeStruct((B, D * PACK), jnp.float32),
        scratch_types=[
            pltpu.VMEM((BPW,), jnp.int32),
            pltpu.VMEM((BPW,), jnp.int32),
            pltpu.VMEM((BPW, D * PACK), jnp.float32),
            pltpu.SemaphoreType.DMA,
        ],
    )
    def k(idx_hbm, t_hbm, g_hbm, idx_v, j_v, rows_v, sem):
        wid = lax.axis_index("s") * NC + lax.axis_index("c")
        base = wid * BPW

        pltpu.sync_copy(idx_hbm.at[pl.ds(base, BPW)], idx_v)

        @pl.loop(0, BPW, step=16)
        def _(i):
            u = idx_v.at[pl.ds(i, 16)][...]
            j_v.at[pl.ds(i, 16)][...] = (
                ((u >> STEPBITS) << BANDBITS) | (u & (BAND - 1))) >> 1

        pltpu.async_copy(t_hbm.at[j_v], rows_v, sem).wait()
        pltpu.sync_copy(rows_v, g_hbm.at[pl.ds(base, BPW)])

    return k(idx, t4)


def _extract(g, idx):
    gi = jax.lax.bitcast_convert_type(g, jnp.int32)
    podd = (idx & 1) == 1
    gsel = jnp.where(podd, gi & jnp.int32(-65536), gi << 16)
    gf = jax.lax.bitcast_convert_type(gsel, jnp.float32).astype(jnp.bfloat16)
    amod = (idx >> BANDBITS) & 3
    out = jnp.zeros((g.shape[0], D), jnp.bfloat16)
    for a in range(PACK):
        m = (amod == a).astype(jnp.bfloat16)
        out = out + m * gf[:, a * D:(a + 1) * D]
    return out


def _mlp_kernel(gu_ref, gp_ref, ui_ref, pi_ref,
                w1u_ref, w1p_ref, b1_ref,
                w2_ref, b2_ref, w3_ref, b3_ref, out_ref):
    ue = _extract(gu_ref[...], ui_ref[...]).astype(jnp.float32)
    pe = _extract(gp_ref[...], pi_ref[...]).astype(jnp.float32)
    h = jnp.dot(ue, w1u_ref[...], preferred_element_type=jnp.float32)
    h = h + jnp.dot(pe, w1p_ref[...], preferred_element_type=jnp.float32)
    h = jnp.maximum(h + b1_ref[...], 0.0)
    h = jnp.maximum(
        jnp.dot(h, w2_ref[...], preferred_element_type=jnp.float32)
        + b2_ref[...], 0.0)
    o = jnp.dot(h, w3_ref[...], preferred_element_type=jnp.float32) + b3_ref[...]
    out_ref[...] = jax.nn.sigmoid(o)


def _mlp_tc(gu, gp, user, product, W1, b1, W2, b2, W3, b3):
    W1uT = W1[:, :D].T
    W1pT = W1[:, D:].T
    W2T = W2.T
    W3T = W3.T
    b1r = b1.reshape(1, 128)
    b2r = b2.reshape(1, 64)
    b3r = b3.reshape(1, 1)
    ui = user.reshape(B, 1)
    pi = product.reshape(B, 1)
    BB = 2048
    grid = (B // BB,)
    return pl.pallas_call(
        _mlp_kernel,
        grid=grid,
        in_specs=[
            pl.BlockSpec((BB, D * PACK), lambda i: (i, 0)),
            pl.BlockSpec((BB, D * PACK), lambda i: (i, 0)),
            pl.BlockSpec((BB, 1), lambda i: (i, 0)),
            pl.BlockSpec((BB, 1), lambda i: (i, 0)),
            pl.BlockSpec((D, 128), lambda i: (0, 0)),
            pl.BlockSpec((D, 128), lambda i: (0, 0)),
            pl.BlockSpec((1, 128), lambda i: (0, 0)),
            pl.BlockSpec((128, 64), lambda i: (0, 0)),
            pl.BlockSpec((1, 64), lambda i: (0, 0)),
            pl.BlockSpec((64, 1), lambda i: (0, 0)),
            pl.BlockSpec((1, 1), lambda i: (0, 0)),
        ],
        out_specs=pl.BlockSpec((BB, 1), lambda i: (i, 0)),
        out_shape=jax.ShapeDtypeStruct((B, 1), jnp.float32),
        compiler_params=pltpu.CompilerParams(
            dimension_semantics=("arbitrary",)),
    )(gu, gp, ui, pi, W1uT, W1pT, b1r, W2T, b2r, W3T, b3r)


def kernel(user, product, user_emb, prod_emb, W1, b1, W2, b2, W3, b3):
    eyes = jnp.eye(D * PACK, dtype=jnp.bfloat16)
    t4u = _pack_table(user_emb.T, eyes)
    gu = _gather_sc(user, t4u)
    t4p = _pack_table(prod_emb.T, eyes)
    gp = _gather_sc(product, t4p)
    return _mlp_tc(gu, gp, user, product, W1, b1, W2, b2, W3, b3)


# speedup vs baseline: 2.7979x; 1.1843x over previous
import functools

import jax
import jax.numpy as jnp
from jax import lax
from jax.experimental import pallas as pl
from jax.experimental.pallas import tpu as pltpu
from jax.experimental.pallas import tpu_sc as plsc

NC = 2
NS = 16
NW = NC * NS
B = 16384
D = 32
V = 1000000
PACK = 4
BPW = B // NW
TC_CHUNK = 32768
BAND = TC_CHUNK // PACK
NSTEP = pl.cdiv(V, TC_CHUNK)
VP = NSTEP * BAND
BANDBITS = BAND.bit_length() - 1
STEPBITS = TC_CHUNK.bit_length() - 1


def _transpose_kernel(x_ref, eyes_ref, o_ref):
    x = x_ref[...].astype(jnp.bfloat16)
    xs = jnp.concatenate(
        [x[:, a * BAND:(a + 1) * BAND] for a in range(PACK)], axis=0)
    acc = jax.lax.dot_general(
        xs, eyes_ref[...],
        dimension_numbers=(((0,), (0,)), ((), ())),
        preferred_element_type=jnp.float32)
    o_ref[...] = pltpu.bitcast(acc.astype(jnp.bfloat16), jnp.float32)


def _pack_table(tabT, eyes):
    return pl.pallas_call(
        _transpose_kernel,
        grid=(NSTEP,),
        in_specs=[
            pl.BlockSpec((D, TC_CHUNK), lambda i: (0, i)),
            pl.BlockSpec((D * PACK, D * PACK), lambda i: (0, 0)),
        ],
        out_specs=pl.BlockSpec((BAND // 2, D * PACK), lambda i: (i, 0)),
        out_shape=jax.ShapeDtypeStruct((VP // 2, D * PACK), jnp.float32),
        compiler_params=pltpu.CompilerParams(
            dimension_semantics=("arbitrary",)),
    )(tabT, eyes)


def _gather_sc(idx, t4):
    mesh = plsc.VectorSubcoreMesh(core_axis_name="c", subcore_axis_name="s")

    @functools.partial(
        pl.kernel,
        mesh=mesh,
        out_type=jax.ShapeDtypeStruct((B, D * PACK), jnp.float32),
        scratch_types=[
            pltpu.VMEM((BPW,), jnp.int32),
            pltpu.VMEM((BPW,), jnp.int32),
            pltpu.VMEM((BPW, D * PACK), jnp.float32),
            pltpu.SemaphoreType.DMA,
        ],
    )
    def k(idx_hbm, t_hbm, g_hbm, idx_v, j_v, rows_v, sem):
        wid = lax.axis_index("s") * NC + lax.axis_index("c")
        base = wid * BPW

        pltpu.sync_copy(idx_hbm.at[pl.ds(base, BPW)], idx_v)

        @pl.loop(0, BPW, step=16)
        def _(i):
            u = idx_v.at[pl.ds(i, 16)][...]
            j_v.at[pl.ds(i, 16)][...] = (
                ((u >> STEPBITS) << BANDBITS) | (u & (BAND - 1))) >> 1

        pltpu.async_copy(t_hbm.at[j_v], rows_v, sem).wait()
        pltpu.sync_copy(rows_v, g_hbm.at[pl.ds(base, BPW)])

    return k(idx, t4)


def _extract(g, idx):
    gi = jax.lax.bitcast_convert_type(g, jnp.int32)
    podd = (idx & 1) == 1
    gsel = jnp.where(podd, gi & jnp.int32(-65536), gi << 16)
    gf = jax.lax.bitcast_convert_type(gsel, jnp.float32).astype(jnp.bfloat16)
    amod = (idx >> BANDBITS) & 3
    out = jnp.zeros((g.shape[0], D), jnp.bfloat16)
    for a in range(PACK):
        m = (amod == a).astype(jnp.bfloat16)
        out = out + m * gf[:, a * D:(a + 1) * D]
    return out


def _mlp_kernel(gu_ref, gp_ref, ui_ref, pi_ref,
                w1u_ref, w1p_ref, b1_ref,
                w2_ref, b2_ref, w3_ref, b3_ref, out_ref):
    ue = _extract(gu_ref[...], ui_ref[...]).astype(jnp.float32)
    pe = _extract(gp_ref[...], pi_ref[...]).astype(jnp.float32)
    h = jnp.dot(ue, w1u_ref[...], preferred_element_type=jnp.float32)
    h = h + jnp.dot(pe, w1p_ref[...], preferred_element_type=jnp.float32)
    h = jnp.maximum(h + b1_ref[...], 0.0)
    h = jnp.maximum(
        jnp.dot(h, w2_ref[...], preferred_element_type=jnp.float32)
        + b2_ref[...], 0.0)
    o = jnp.dot(h, w3_ref[...], preferred_element_type=jnp.float32) + b3_ref[...]
    out_ref[...] = jax.nn.sigmoid(o)


def _mlp_tc(gu, gp, user, product, W1, b1, W2, b2, W3, b3):
    W1uT = W1[:, :D].T
    W1pT = W1[:, D:].T
    W2T = W2.T
    W3T = W3.T
    b1r = b1.reshape(1, 128)
    b2r = b2.reshape(1, 64)
    b3r = b3.reshape(1, 1)
    ui = user.reshape(B, 1)
    pi = product.reshape(B, 1)
    BB = 4096
    grid = (B // BB,)
    return pl.pallas_call(
        _mlp_kernel,
        grid=grid,
        in_specs=[
            pl.BlockSpec((BB, D * PACK), lambda i: (i, 0)),
            pl.BlockSpec((BB, D * PACK), lambda i: (i, 0)),
            pl.BlockSpec((BB, 1), lambda i: (i, 0)),
            pl.BlockSpec((BB, 1), lambda i: (i, 0)),
            pl.BlockSpec((D, 128), lambda i: (0, 0)),
            pl.BlockSpec((D, 128), lambda i: (0, 0)),
            pl.BlockSpec((1, 128), lambda i: (0, 0)),
            pl.BlockSpec((128, 64), lambda i: (0, 0)),
            pl.BlockSpec((1, 64), lambda i: (0, 0)),
            pl.BlockSpec((64, 1), lambda i: (0, 0)),
            pl.BlockSpec((1, 1), lambda i: (0, 0)),
        ],
        out_specs=pl.BlockSpec((BB, 1), lambda i: (i, 0)),
        out_shape=jax.ShapeDtypeStruct((B, 1), jnp.float32),
        compiler_params=pltpu.CompilerParams(
            dimension_semantics=("arbitrary",)),
    )(gu, gp, ui, pi, W1uT, W1pT, b1r, W2T, b2r, W3T, b3r)


def kernel(user, product, user_emb, prod_emb, W1, b1, W2, b2, W3, b3):
    eyes = jnp.eye(D * PACK, dtype=jnp.bfloat16)
    t4u = _pack_table(user_emb.T, eyes)
    gu = _gather_sc(user, t4u)
    t4p = _pack_table(prod_emb.T, eyes)
    gp = _gather_sc(product, t4p)
    return _mlp_tc(gu, gp, user, product, W1, b1, W2, b2, W3, b3)
